# Initial kernel scaffold; baseline (speedup 1.0000x reference)
import jax
import jax.numpy as jnp
from jax.experimental import pallas as pl


def kernel(x, pos, edge_index, W_in, b_in, W1, b1, W2, b2, W3, b3, W4, b4, W_o1, b_o1, W_o2, b_o2):
    raise NotImplementedError("write your pallas kernel here")


# speedup vs baseline: 4.7138x; 4.7138x over previous
import functools

import jax
import jax.numpy as jnp
from jax import lax
from jax.experimental import pallas as pl
from jax.experimental.pallas import tpu as pltpu
from jax.experimental.pallas import tpu_sc as plsc

N = 10000
E = 160000
WIDTH = 256
HALF = 128
NSUB = 16
CHUNK = 128
NCHUNK = 80
PER = CHUNK * NCHUNK
ZROWS = 632
ACC_ROWS = NSUB * ZROWS
WLAST = N - 15 * ZROWS


def _deg_body(dst_hbm, ones_hbm, zeros_hbm, deg_out, dst_buf, ones_buf, acc):
    c = lax.axis_index("c")
    s = lax.axis_index("s")

    @pl.when(c == 0)
    def _():
        pltpu.sync_copy(dst_hbm.at[s], dst_buf)
        pltpu.sync_copy(ones_hbm, ones_buf)
        pltpu.sync_copy(zeros_hbm, acc.at[pl.ds(s * ZROWS, ZROWS)])
        plsc.subcore_barrier()

        def step(i, carry):
            pltpu.sync_copy(ones_buf, acc.at[dst_buf.at[i]], add=True)
            return carry

        lax.fori_loop(0, NCHUNK, step, 0)
        plsc.subcore_barrier()
        _writeback(acc, deg_out, s)


def _writeback(acc, out, sub):
    @pl.when(sub < NSUB - 1)
    def _():
        pltpu.sync_copy(acc.at[pl.ds(sub * ZROWS, ZROWS)],
                        out.at[pl.ds(sub * ZROWS, ZROWS)])

    @pl.when(sub == NSUB - 1)
    def _():
        pltpu.sync_copy(acc.at[pl.ds((NSUB - 1) * ZROWS, WLAST)],
                        out.at[pl.ds((NSUB - 1) * ZROWS, WLAST)])


def _agg_run(u, s_out, src_buf, dst_buf, buf0, acc, sem0, sub):
    def step(i, carry):
        pltpu.async_copy(u.at[src_buf.at[i]], buf0, sem0).wait()
        pltpu.sync_copy(buf0, acc.at[dst_buf.at[i]], add=True)
        return carry

    lax.fori_loop(0, NCHUNK, step, 0)
    plsc.subcore_barrier()
    _writeback(acc, s_out, sub)


def _agg_body(u_lo, u_hi, src_hbm, dst_hbm, zeros_hbm, s_lo, s_hi,
              src_buf, dst_buf, buf0, acc, sem0):
    c = lax.axis_index("c")
    s = lax.axis_index("s")
    pltpu.sync_copy(src_hbm.at[s], src_buf)
    pltpu.sync_copy(dst_hbm.at[s], dst_buf)
    pltpu.sync_copy(zeros_hbm, acc.at[pl.ds(s * ZROWS, ZROWS)])
    plsc.subcore_barrier()

    @pl.when(c == 0)
    def _():
        _agg_run(u_lo, s_lo, src_buf, dst_buf, buf0, acc, sem0, s)

    @pl.when(c == 1)
    def _():
        _agg_run(u_hi, s_hi, src_buf, dst_buf, buf0, acc, sem0, s)


@functools.cache
def _sc_kernels():
    mesh = plsc.VectorSubcoreMesh(core_axis_name="c", subcore_axis_name="s",
                                  num_cores=2, num_subcores=NSUB)
    deg_k = pl.kernel(
        _deg_body,
        out_type=jax.ShapeDtypeStruct((N, HALF), jnp.float32),
        mesh=mesh,
        scratch_types=[
            pltpu.VMEM((NCHUNK, CHUNK), jnp.int32),
            pltpu.VMEM((CHUNK, HALF), jnp.float32),
            pltpu.VMEM_SHARED((ACC_ROWS, HALF), jnp.float32),
        ],
    )
    agg_k = pl.kernel(
        _agg_body,
        out_type=(jax.ShapeDtypeStruct((N, HALF), jnp.float32),
                  jax.ShapeDtypeStruct((N, HALF), jnp.float32)),
        mesh=mesh,
        scratch_types=[
            pltpu.VMEM((NCHUNK, CHUNK), jnp.int32),
            pltpu.VMEM((NCHUNK, CHUNK), jnp.int32),
            pltpu.VMEM((CHUNK, HALF), jnp.float32),
            pltpu.VMEM_SHARED((ACC_ROWS, HALF), jnp.float32),
            pltpu.SemaphoreType.DMA,
        ],
    )
    return deg_k, agg_k


ROWS = 400
GRID = N // ROWS


def _init_body(cat_ref, win_ref, bin_ref, w1_ref, deg_ref,
               h0_ref, ulo_ref, uhi_ref):
    h0 = jnp.dot(cat_ref[...], win_ref[...],
                 preferred_element_type=jnp.float32,
                 precision=lax.Precision.HIGHEST) + bin_ref[...]
    h0_ref[...] = h0
    dinv = lax.rsqrt(deg_ref[...] + 1.0)
    hw = jnp.dot(h0, w1_ref[...], preferred_element_type=jnp.float32,
                 precision=lax.Precision.HIGHEST)
    ulo_ref[...] = hw[:, :HALF] * dinv
    uhi_ref[...] = hw[:, HALF:] * dinv


def _layer_body(h_ref, slo_ref, shi_ref, ulo_ref, uhi_ref, deg_ref,
                w_ref, b_ref, hn_ref, ulon_ref, uhin_ref):
    dinv = lax.rsqrt(deg_ref[...] + 1.0)
    glo = (slo_ref[...] + ulo_ref[...]) * dinv
    ghi = (shi_ref[...] + uhi_ref[...]) * dinv
    g = jnp.concatenate([glo, ghi], axis=1) + b_ref[...]
    hn = jnp.maximum(h_ref[...] + g, 0.0)
    hn_ref[...] = hn
    hw = jnp.dot(hn, w_ref[...], preferred_element_type=jnp.float32,
                 precision=lax.Precision.HIGHEST)
    ulon_ref[...] = hw[:, :HALF] * dinv
    uhin_ref[...] = hw[:, HALF:] * dinv


def _head_body(h_ref, slo_ref, shi_ref, ulo_ref, uhi_ref, deg_ref,
               b4_ref, wo1_ref, bo1_ref, wo2_ref, bo2_ref, out_ref):
    dinv = lax.rsqrt(deg_ref[...] + 1.0)
    glo = (slo_ref[...] + ulo_ref[...]) * dinv
    ghi = (shi_ref[...] + uhi_ref[...]) * dinv
    g = jnp.concatenate([glo, ghi], axis=1) + b4_ref[...]
    h16 = jnp.maximum(h_ref[...] + g, 0.0)
    t = jnp.dot(h16, wo1_ref[...], preferred_element_type=jnp.float32,
                 precision=lax.Precision.HIGHEST) + bo1_ref[...]
    t = jnp.where(t >= 0.0, t, 0.01 * t)
    out_ref[...] = jnp.dot(t, wo2_ref[...],
                           preferred_element_type=jnp.float32,
                 precision=lax.Precision.HIGHEST) + bo2_ref[...]


def _row_spec(w):
    return pl.BlockSpec((ROWS, w), lambda i: (i, 0))


def _full_spec(r, c):
    return pl.BlockSpec((r, c), lambda i: (0, 0))


_init_call = pl.pallas_call(
    _init_body,
    grid=(GRID,),
    in_specs=[_row_spec(HALF), _full_spec(HALF, WIDTH), _full_spec(1, WIDTH),
              _full_spec(WIDTH, WIDTH), _row_spec(HALF)],
    out_specs=[_row_spec(WIDTH), _row_spec(HALF), _row_spec(HALF)],
    out_shape=[jax.ShapeDtypeStruct((N, WIDTH), jnp.float32),
               jax.ShapeDtypeStruct((N, HALF), jnp.float32),
               jax.ShapeDtypeStruct((N, HALF), jnp.float32)],
)

_layer_call = pl.pallas_call(
    _layer_body,
    grid=(GRID,),
    in_specs=[_row_spec(WIDTH), _row_spec(HALF), _row_spec(HALF),
              _row_spec(HALF), _row_spec(HALF), _row_spec(HALF),
              _full_spec(WIDTH, WIDTH), _full_spec(1, WIDTH)],
    out_specs=[_row_spec(WIDTH), _row_spec(HALF), _row_spec(HALF)],
    out_shape=[jax.ShapeDtypeStruct((N, WIDTH), jnp.float32),
               jax.ShapeDtypeStruct((N, HALF), jnp.float32),
               jax.ShapeDtypeStruct((N, HALF), jnp.float32)],
)

_head_call = pl.pallas_call(
    _head_body,
    grid=(GRID,),
    in_specs=[_row_spec(WIDTH), _row_spec(HALF), _row_spec(HALF),
              _row_spec(HALF), _row_spec(HALF), _row_spec(HALF),
              _full_spec(1, WIDTH), _full_spec(WIDTH, WIDTH),
              _full_spec(1, WIDTH), _full_spec(WIDTH, HALF),
              _full_spec(1, HALF)],
    out_specs=_row_spec(HALF),
    out_shape=jax.ShapeDtypeStruct((N, HALF), jnp.float32),
)


def kernel(x, pos, edge_index, W_in, b_in, W1, b1, W2, b2, W3, b3, W4, b4,
           W_o1, b_o1, W_o2, b_o2):
    cat = jnp.concatenate([pos, x], axis=1)
    cat_pad = jnp.pad(cat, ((0, 0), (0, HALF - 3)))
    win_pad = jnp.pad(W_in, ((0, HALF - 3), (0, 0)))
    wo2_pad = jnp.pad(W_o2, ((0, 0), (0, HALF - 1)))
    bo2_pad = jnp.pad(b_o2, (0, HALF - 1))

    src = jnp.pad(edge_index[0], (0, NSUB * PER - E)).reshape(NSUB, NCHUNK, CHUNK)
    dst = jnp.pad(edge_index[1], (0, NSUB * PER - E),
                  constant_values=N).reshape(NSUB, NCHUNK, CHUNK)

    zeros_hbm = jnp.zeros((ZROWS, HALF), jnp.float32)
    ones_hbm = jnp.ones((CHUNK, HALF), jnp.float32)

    deg_k, agg_k = _sc_kernels()
    deg = deg_k(dst, ones_hbm, zeros_hbm)

    h, ulo, uhi = _init_call(cat_pad, win_pad, b_in[None], W1, deg)

    ws = [W1, W2, W3, W4]
    bs = [b1, b2, b3, b4]
    for t in range(16):
        slo, shi = agg_k(ulo, uhi, src, dst, zeros_hbm)
        b_cur = bs[t % 4][None]
        if t < 15:
            h, ulo, uhi = _layer_call(h, slo, shi, ulo, uhi, deg,
                                      ws[(t + 1) % 4], b_cur)
        else:
            res = _head_call(h, slo, shi, ulo, uhi, deg, b_cur,
                             W_o1, b_o1[None], wo2_pad, bo2_pad[None])
    return res[:, :1]
